# baseline (speedup 1.0000x reference)
import functools

import jax
import jax.numpy as jnp
from jax import lax
from jax.experimental import pallas as pl
from jax.experimental.pallas import tpu as pltpu
from jax.experimental.pallas import tpu_sc as plsc

_C = 1.0
_R = 2.0
_T = 1.0
_MIN_NORM = 1e-15
_MAXNORM = (1.0 - 4e-3) / (_C ** 0.5)

_L = 16
_D = 128
_BC = 200
_WB = 10
_NB = 3
_NC = 2
_NS = 16
_NW = _NC * _NS
_SPLIT = (104, 96)


@functools.lru_cache(maxsize=None)
def _make_sc_dots(n_nodes, n_edges):
    assert n_edges % (_NW * _BC * _WB) == 0
    cpw = n_edges // (_NW * _BC)
    ntm = (cpw - (_NB - 1)) // _NB
    tail_slots = list(range(ntm * _NB, cpw))
    ngrp = -(-_BC // _L)
    mesh = plsc.VectorSubcoreMesh(core_axis_name="c", subcore_axis_name="s")
    f32 = jnp.float32

    @functools.partial(
        pl.kernel,
        mesh=mesh,
        compiler_params=pltpu.CompilerParams(needs_layout_passes=False),
        out_type=[jax.ShapeDtypeStruct((n_edges,), f32)] * 3,
        scratch_types=[
            pltpu.VMEM((_BC,), jnp.int32),
            pltpu.VMEM((_BC,), jnp.int32),
            pltpu.VMEM((_BC,), jnp.int32),
            pltpu.VMEM((_BC,), jnp.int32),
            pltpu.VMEM((_BC,), jnp.int32),
            pltpu.VMEM((_BC,), jnp.int32),
            pltpu.VMEM((_BC, _D), f32),
            pltpu.VMEM((_BC, _D), f32),
            pltpu.VMEM((_BC, _D), f32),
            pltpu.VMEM((n_nodes,), f32),
            pltpu.VMEM((_WB * _BC,), f32),
            pltpu.VMEM((_WB * _BC,), f32),
            pltpu.VMEM((_WB * _BC,), f32),
            pltpu.VMEM((_L, _L + 1), f32),
            pltpu.VMEM((_L, _L + 1), f32),
            pltpu.SemaphoreType.DMA,
            pltpu.SemaphoreType.DMA,
            pltpu.SemaphoreType.DMA,
        ],
    )
    def sc_dots(h_hbm, i0_hbm, i1_hbm, sqn_hbm, xx_hbm, yy_hbm, xy_hbm,
                i0a, i0b, i0c, i1a, i1b, i1c, rsa, rsb, rsc, sqn_v,
                xx_v, yy_v, xy_v, ps0_v, ps1_v, sem0, sem1, sem2):
        wid = lax.axis_index("s") * _NC + lax.axis_index("c")
        sems = (sem0, sem1, sem2)
        i0_bufs = (i0a, i0b, i0c)
        i1_bufs = (i1a, i1b, i1c)
        rs_bufs = (rsa, rsb, rsc)
        c0 = wid * cpw

        pltpu.sync_copy(sqn_hbm, sqn_v)

        def copies(b, side):
            idx = (i0_bufs if side == 0 else i1_bufs)[b]
            cps = []
            off = 0
            for w in _SPLIT:
                cps.append(pltpu.make_async_copy(
                    h_hbm.at[idx.at[pl.ds(off, w)]],
                    rs_bufs[b].at[pl.ds(off, w)], sems[b]))
                off += w
            return cps

        def fire_w(b, slot):
            cidx = c0 + slot
            pltpu.sync_copy(i0_hbm.at[pl.ds(cidx * _BC, _BC)], i0_bufs[b])
            pltpu.sync_copy(i1_hbm.at[pl.ds(cidx * _BC, _BC)], i1_bufs[b])
            for cp in copies(b, 0):
                cp.start()

        def wait(b, side):
            for cp in copies(b, side):
                cp.wait()

        def fire_a(b):
            for cp in copies(b, 1):
                cp.start(add=True)

        def compute(b, slot):
            sbase = (slot % _WB) * _BC
            rs_v = rs_bufs[b]
            lanes = lax.iota(jnp.int32, _L)

            def acc_phase(base16, ps_ref):
                accs = [jnp.zeros((_L,), f32)] * _L
                for k in range(_D // _L):
                    for e in range(_L):
                        sv = rs_v[base16 + e, pl.ds(k * _L, _L)]
                        accs[e] = accs[e] + sv * sv
                for e in range(_L):
                    ps_ref[e, pl.ds(0, _L)] = accs[e]

            def red_phase(base16, ps_ref):
                parts = [jnp.zeros((_L,), f32) for _ in range(4)]
                for j in range(4):
                    for m in range(4):
                        col = jnp.full((_L,), j * 4 + m, jnp.int32)
                        parts[j] = parts[j] + plsc.load_gather(
                            ps_ref, [lanes, col])
                s2 = (parts[0] + parts[1]) + (parts[2] + parts[3])
                i0l = i0_bufs[b][pl.ds(base16, _L)]
                i1l = i1_bufs[b][pl.ds(base16, _L)]
                xx = plsc.load_gather(sqn_v, [i0l])
                yy = plsc.load_gather(sqn_v, [i1l])
                xy = 0.5 * (s2 - xx - yy)
                dst = sbase + base16
                xx_v[pl.ds(dst, _L)] = xx
                yy_v[pl.ds(dst, _L)] = yy
                xy_v[pl.ds(dst, _L)] = xy

            def pair_body(gg, carry):
                b0 = gg * (2 * _L)
                b1 = b0 + _L
                acc_phase(b0, ps0_v)
                acc_phase(b1, ps1_v)
                red_phase(b0, ps0_v)
                red_phase(b1, ps1_v)
                return carry

            lax.fori_loop(0, ngrp // 2, pair_body, 0)
            acc_phase(_BC - _L, ps0_v)
            red_phase(_BC - _L, ps0_v)

            @pl.when(slot % _WB == _WB - 1)
            def _():
                ebase = (c0 + slot - (_WB - 1)) * _BC
                pltpu.sync_copy(xx_v, xx_hbm.at[pl.ds(ebase, _WB * _BC)])
                pltpu.sync_copy(yy_v, yy_hbm.at[pl.ds(ebase, _WB * _BC)])
                pltpu.sync_copy(xy_v, xy_hbm.at[pl.ds(ebase, _WB * _BC)])

        def step(slot, b):
            bw = (b + 2) % _NB
            ba = (b + 1) % _NB

            @pl.when(slot + 2 < cpw)
            def _():
                fire_w(bw, slot + 2)

            @pl.when(slot + 1 < cpw)
            def _():
                wait(ba, 0)
                fire_a(ba)

            wait(b, 1)
            compute(b, slot)

        fire_w(0, 0)
        wait(0, 0)
        fire_a(0)
        fire_w(1, 1)

        def t_body(tm, carry):
            for b in range(_NB):
                step(tm * _NB + b, b)
            return carry

        lax.fori_loop(0, ntm, t_body, 0)
        for slot in tail_slots:
            step(slot, slot % _NB)

    return sc_dots


def _sqn_body(h_ref, o_ref):
    x = h_ref[...]
    o_ref[...] = jnp.sum(x * x, axis=1, keepdims=True)


@functools.lru_cache(maxsize=None)
def _make_sqn(n_nodes):
    return pl.pallas_call(
        _sqn_body,
        out_shape=jax.ShapeDtypeStruct((n_nodes, 1), jnp.float32),
    )


def _tail_body(xx_ref, yy_ref, xy_ref, o_ref):
    xx = xx_ref[...]
    yy = yy_ref[...]
    xy = xy_ref[...]
    nx = jnp.maximum(jnp.sqrt(xx), _MIN_NORM)
    ny = jnp.maximum(jnp.sqrt(yy), _MIN_NORM)
    fx = jnp.where(nx > _MAXNORM, _MAXNORM / nx, 1.0)
    fy = jnp.where(ny > _MAXNORM, _MAXNORM / ny, 1.0)
    x2 = fx * fx * xx
    y2 = fy * fy * yy
    pxy = fx * fy * xy
    a = 1.0 - 2.0 * _C * pxy + _C * y2
    b = 1.0 - _C * x2
    num2 = a * a * x2 + b * b * y2 - 2.0 * a * b * pxy
    den = jnp.maximum(1.0 - 2.0 * _C * pxy + _C * _C * x2 * y2, _MIN_NORM)
    sqrt_c = _C ** 0.5
    z = sqrt_c * jnp.sqrt(jnp.maximum(num2, 0.0)) / den
    zc = jnp.clip(z, -1.0 + 1e-7, 1.0 - 1e-7)
    dist_c = 0.5 * jnp.log((1.0 + zc) / (1.0 - zc))
    dist = dist_c * 2.0 / sqrt_c
    sqdist = dist * dist
    o_ref[...] = 1.0 / (jnp.exp((sqdist - _R) / _T) + 1.0)


@functools.lru_cache(maxsize=None)
def _make_tail(n_edges):
    rows = n_edges // _D
    return pl.pallas_call(
        _tail_body,
        out_shape=jax.ShapeDtypeStruct((rows, _D), jnp.float32),
    )


def kernel(h, idx):
    n_nodes, d = h.shape
    n_edges = idx.shape[0]
    assert d == _D
    i0 = idx[:, 0]
    i1 = idx[:, 1]
    sqn = _make_sqn(n_nodes)(h).reshape(n_nodes)
    xx, yy, xy = _make_sc_dots(n_nodes, n_edges)(h, i0, i1, sqn)
    probs = _make_tail(n_edges)(
        xx.reshape(-1, _D), yy.reshape(-1, _D), xy.reshape(-1, _D))
    return probs.reshape(n_edges)


# speedup vs baseline: 1.0158x; 1.0158x over previous
import functools

import jax
import jax.numpy as jnp
from jax import lax
from jax.experimental import pallas as pl
from jax.experimental.pallas import tpu as pltpu
from jax.experimental.pallas import tpu_sc as plsc

_C = 1.0
_R = 2.0
_T = 1.0
_MIN_NORM = 1e-15
_MAXNORM = (1.0 - 4e-3) / (_C ** 0.5)

_L = 16
_D = 128
_BC = 200
_WB = 10
_NB = 3
_NC = 2
_NS = 16
_NW = _NC * _NS
_SPLIT = (104, 96)


@functools.lru_cache(maxsize=None)
def _make_sc_dots(n_nodes, n_edges):
    assert n_edges % (_NW * _BC * _WB) == 0
    cpw = n_edges // (_NW * _BC)
    ntm = (cpw - (_NB - 1)) // _NB
    tail_slots = list(range(ntm * _NB, cpw))
    ngrp = -(-_BC // _L)
    mesh = plsc.VectorSubcoreMesh(core_axis_name="c", subcore_axis_name="s")
    f32 = jnp.float32

    @functools.partial(
        pl.kernel,
        mesh=mesh,
        compiler_params=pltpu.CompilerParams(needs_layout_passes=False),
        out_type=[jax.ShapeDtypeStruct((n_edges,), f32)] * 3,
        scratch_types=[
            pltpu.VMEM((_BC,), jnp.int32),
            pltpu.VMEM((_BC,), jnp.int32),
            pltpu.VMEM((_BC,), jnp.int32),
            pltpu.VMEM((_BC,), jnp.int32),
            pltpu.VMEM((_BC,), jnp.int32),
            pltpu.VMEM((_BC,), jnp.int32),
            pltpu.VMEM((_BC, _D), f32),
            pltpu.VMEM((_BC, _D), f32),
            pltpu.VMEM((_BC, _D), f32),
            pltpu.VMEM((n_nodes,), f32),
            pltpu.VMEM((_WB * _BC,), f32),
            pltpu.VMEM((_WB * _BC,), f32),
            pltpu.VMEM((_WB * _BC,), f32),
            pltpu.VMEM((_L, _L + 1), f32),
            pltpu.VMEM((_L, _L + 1), f32),
            pltpu.SemaphoreType.DMA,
            pltpu.SemaphoreType.DMA,
            pltpu.SemaphoreType.DMA,
        ],
    )
    def sc_dots(h_hbm, i0_hbm, i1_hbm, sqn_hbm, xx_hbm, yy_hbm, xy_hbm,
                i0a, i0b, i0c, i1a, i1b, i1c, rsa, rsb, rsc, sqn_v,
                xx_v, yy_v, xy_v, ps0_v, ps1_v, sem0, sem1, sem2):
        wid = lax.axis_index("s") * _NC + lax.axis_index("c")
        sems = (sem0, sem1, sem2)
        i0_bufs = (i0a, i0b, i0c)
        i1_bufs = (i1a, i1b, i1c)
        rs_bufs = (rsa, rsb, rsc)
        c0 = wid * cpw

        pltpu.sync_copy(sqn_hbm, sqn_v)

        def copies(b, side):
            idx = (i0_bufs if side == 0 else i1_bufs)[b]
            cps = []
            off = 0
            for w in _SPLIT:
                cps.append(pltpu.make_async_copy(
                    h_hbm.at[idx.at[pl.ds(off, w)]],
                    rs_bufs[b].at[pl.ds(off, w)], sems[b]))
                off += w
            return cps

        def fire_w(b, slot):
            cidx = c0 + slot
            pltpu.sync_copy(i0_hbm.at[pl.ds(cidx * _BC, _BC)], i0_bufs[b])
            pltpu.sync_copy(i1_hbm.at[pl.ds(cidx * _BC, _BC)], i1_bufs[b])
            for cp in copies(b, 0):
                cp.start()

        def wait(b, side):
            for cp in copies(b, side):
                cp.wait()

        def fire_a(b):
            for cp in copies(b, 1):
                cp.start(add=True)

        def compute(b, slot):
            sbase = (slot % _WB) * _BC
            rs_v = rs_bufs[b]
            lanes = lax.iota(jnp.int32, _L)

            def acc_phase(base16, ps_ref):
                accs = [jnp.zeros((_L,), f32)] * _L
                for k in range(_D // _L):
                    for e in range(_L):
                        sv = rs_v[base16 + e, pl.ds(k * _L, _L)]
                        accs[e] = accs[e] + sv * sv
                for e in range(_L):
                    ps_ref[e, pl.ds(0, _L)] = accs[e]

            def red_phase(base16, ps_ref):
                parts = [jnp.zeros((_L,), f32) for _ in range(4)]
                for j in range(4):
                    for m in range(4):
                        col = jnp.full((_L,), j * 4 + m, jnp.int32)
                        parts[j] = parts[j] + plsc.load_gather(
                            ps_ref, [lanes, col])
                s2 = (parts[0] + parts[1]) + (parts[2] + parts[3])
                i0l = i0_bufs[b][pl.ds(base16, _L)]
                i1l = i1_bufs[b][pl.ds(base16, _L)]
                xx = plsc.load_gather(sqn_v, [i0l])
                yy = plsc.load_gather(sqn_v, [i1l])
                xy = 0.5 * (s2 - xx - yy)
                dst = sbase + base16
                xx_v[pl.ds(dst, _L)] = xx
                yy_v[pl.ds(dst, _L)] = yy
                xy_v[pl.ds(dst, _L)] = xy

            def group_body(g, carry):
                base16 = jnp.minimum(g * _L, _BC - _L)
                acc_phase(base16, ps0_v)
                red_phase(base16, ps0_v)
                return carry

            lax.fori_loop(0, ngrp, group_body, 0)

            @pl.when(slot % _WB == _WB - 1)
            def _():
                ebase = (c0 + slot - (_WB - 1)) * _BC
                pltpu.sync_copy(xx_v, xx_hbm.at[pl.ds(ebase, _WB * _BC)])
                pltpu.sync_copy(yy_v, yy_hbm.at[pl.ds(ebase, _WB * _BC)])
                pltpu.sync_copy(xy_v, xy_hbm.at[pl.ds(ebase, _WB * _BC)])

        def step(slot, b):
            bw = (b + 2) % _NB
            ba = (b + 1) % _NB

            @pl.when(slot + 2 < cpw)
            def _():
                fire_w(bw, slot + 2)

            @pl.when(slot + 1 < cpw)
            def _():
                wait(ba, 0)
                fire_a(ba)

            wait(b, 1)
            compute(b, slot)

        fire_w(0, 0)
        wait(0, 0)
        fire_a(0)
        fire_w(1, 1)

        def t_body(tm, carry):
            for b in range(_NB):
                step(tm * _NB + b, b)
            return carry

        lax.fori_loop(0, ntm, t_body, 0)
        for slot in tail_slots:
            step(slot, slot % _NB)

    return sc_dots


def _sqn_body(h_ref, o_ref):
    x = h_ref[...]
    o_ref[...] = jnp.sum(x * x, axis=1, keepdims=True)


@functools.lru_cache(maxsize=None)
def _make_sqn(n_nodes):
    return pl.pallas_call(
        _sqn_body,
        out_shape=jax.ShapeDtypeStruct((n_nodes, 1), jnp.float32),
    )


def _tail_body(xx_ref, yy_ref, xy_ref, o_ref):
    xx = xx_ref[...]
    yy = yy_ref[...]
    xy = xy_ref[...]
    nx = jnp.maximum(jnp.sqrt(xx), _MIN_NORM)
    ny = jnp.maximum(jnp.sqrt(yy), _MIN_NORM)
    fx = jnp.where(nx > _MAXNORM, _MAXNORM / nx, 1.0)
    fy = jnp.where(ny > _MAXNORM, _MAXNORM / ny, 1.0)
    x2 = fx * fx * xx
    y2 = fy * fy * yy
    pxy = fx * fy * xy
    a = 1.0 - 2.0 * _C * pxy + _C * y2
    b = 1.0 - _C * x2
    num2 = a * a * x2 + b * b * y2 - 2.0 * a * b * pxy
    den = jnp.maximum(1.0 - 2.0 * _C * pxy + _C * _C * x2 * y2, _MIN_NORM)
    sqrt_c = _C ** 0.5
    z = sqrt_c * jnp.sqrt(jnp.maximum(num2, 0.0)) / den
    zc = jnp.clip(z, -1.0 + 1e-7, 1.0 - 1e-7)
    dist_c = 0.5 * jnp.log((1.0 + zc) / (1.0 - zc))
    dist = dist_c * 2.0 / sqrt_c
    sqdist = dist * dist
    o_ref[...] = 1.0 / (jnp.exp((sqdist - _R) / _T) + 1.0)


@functools.lru_cache(maxsize=None)
def _make_tail(n_edges):
    rows = n_edges // _D
    return pl.pallas_call(
        _tail_body,
        out_shape=jax.ShapeDtypeStruct((rows, _D), jnp.float32),
    )


def kernel(h, idx):
    n_nodes, d = h.shape
    n_edges = idx.shape[0]
    assert d == _D
    i0 = idx[:, 0]
    i1 = idx[:, 1]
    sqn = _make_sqn(n_nodes)(h).reshape(n_nodes)
    xx, yy, xy = _make_sc_dots(n_nodes, n_edges)(h, i0, i1, sqn)
    probs = _make_tail(n_edges)(
        xx.reshape(-1, _D), yy.reshape(-1, _D), xy.reshape(-1, _D))
    return probs.reshape(n_edges)


# speedup vs baseline: 1.0172x; 1.0014x over previous
import functools

import jax
import jax.numpy as jnp
from jax import lax
from jax.experimental import pallas as pl
from jax.experimental.pallas import tpu as pltpu
from jax.experimental.pallas import tpu_sc as plsc

_C = 1.0
_R = 2.0
_T = 1.0
_MIN_NORM = 1e-15
_MAXNORM = (1.0 - 4e-3) / (_C ** 0.5)

_L = 16
_D = 128
_BC = 200
_WB = 10
_NB = 3
_NC = 2
_NS = 16
_NW = _NC * _NS
_SPLIT = (128, 72)


@functools.lru_cache(maxsize=None)
def _make_sc_dots(n_nodes, n_edges):
    assert n_edges % (_NW * _BC * _WB) == 0
    cpw = n_edges // (_NW * _BC)
    ntm = (cpw - (_NB - 1)) // _NB
    tail_slots = list(range(ntm * _NB, cpw))
    ngrp = -(-_BC // _L)
    mesh = plsc.VectorSubcoreMesh(core_axis_name="c", subcore_axis_name="s")
    f32 = jnp.float32

    @functools.partial(
        pl.kernel,
        mesh=mesh,
        compiler_params=pltpu.CompilerParams(needs_layout_passes=False),
        out_type=[jax.ShapeDtypeStruct((n_edges,), f32)] * 3,
        scratch_types=[
            pltpu.VMEM((_BC,), jnp.int32),
            pltpu.VMEM((_BC,), jnp.int32),
            pltpu.VMEM((_BC,), jnp.int32),
            pltpu.VMEM((_BC,), jnp.int32),
            pltpu.VMEM((_BC,), jnp.int32),
            pltpu.VMEM((_BC,), jnp.int32),
            pltpu.VMEM((_BC, _D), f32),
            pltpu.VMEM((_BC, _D), f32),
            pltpu.VMEM((_BC, _D), f32),
            pltpu.VMEM((n_nodes,), f32),
            pltpu.VMEM((_WB * _BC,), f32),
            pltpu.VMEM((_WB * _BC,), f32),
            pltpu.VMEM((_WB * _BC,), f32),
            pltpu.VMEM((_L, _L + 1), f32),
            pltpu.VMEM((_L, _L + 1), f32),
            pltpu.SemaphoreType.DMA,
            pltpu.SemaphoreType.DMA,
            pltpu.SemaphoreType.DMA,
        ],
    )
    def sc_dots(h_hbm, i0_hbm, i1_hbm, sqn_hbm, xx_hbm, yy_hbm, xy_hbm,
                i0a, i0b, i0c, i1a, i1b, i1c, rsa, rsb, rsc, sqn_v,
                xx_v, yy_v, xy_v, ps0_v, ps1_v, sem0, sem1, sem2):
        wid = lax.axis_index("s") * _NC + lax.axis_index("c")
        sems = (sem0, sem1, sem2)
        i0_bufs = (i0a, i0b, i0c)
        i1_bufs = (i1a, i1b, i1c)
        rs_bufs = (rsa, rsb, rsc)
        c0 = wid * cpw

        pltpu.sync_copy(sqn_hbm, sqn_v)

        def copies(b, side):
            idx = (i0_bufs if side == 0 else i1_bufs)[b]
            cps = []
            off = 0
            for w in _SPLIT:
                cps.append(pltpu.make_async_copy(
                    h_hbm.at[idx.at[pl.ds(off, w)]],
                    rs_bufs[b].at[pl.ds(off, w)], sems[b]))
                off += w
            return cps

        def fire_w(b, slot):
            cidx = c0 + slot
            pltpu.sync_copy(i0_hbm.at[pl.ds(cidx * _BC, _BC)], i0_bufs[b])
            pltpu.sync_copy(i1_hbm.at[pl.ds(cidx * _BC, _BC)], i1_bufs[b])
            for cp in copies(b, 0):
                cp.start()

        def wait(b, side):
            for cp in copies(b, side):
                cp.wait()

        def fire_a(b):
            for cp in copies(b, 1):
                cp.start(add=True)

        def compute(b, slot):
            sbase = (slot % _WB) * _BC
            rs_v = rs_bufs[b]
            lanes = lax.iota(jnp.int32, _L)

            def acc_phase(base16, ps_ref):
                accs = [None] * _L
                for k in range(_D // _L):
                    for e in range(_L):
                        sv = rs_v[base16 + e, pl.ds(k * _L, _L)]
                        sq = sv * sv
                        accs[e] = sq if k == 0 else accs[e] + sq
                for e in range(_L):
                    ps_ref[e, pl.ds(0, _L)] = accs[e]

            def red_phase(base16, ps_ref):
                parts = [None] * 4
                for j in range(4):
                    for m in range(4):
                        col = jnp.full((_L,), j * 4 + m, jnp.int32)
                        g16 = plsc.load_gather(ps_ref, [lanes, col])
                        parts[j] = g16 if m == 0 else parts[j] + g16
                s2 = (parts[0] + parts[1]) + (parts[2] + parts[3])
                i0l = i0_bufs[b][pl.ds(base16, _L)]
                i1l = i1_bufs[b][pl.ds(base16, _L)]
                xx = plsc.load_gather(sqn_v, [i0l])
                yy = plsc.load_gather(sqn_v, [i1l])
                xy = 0.5 * (s2 - xx - yy)
                dst = sbase + base16
                xx_v[pl.ds(dst, _L)] = xx
                yy_v[pl.ds(dst, _L)] = yy
                xy_v[pl.ds(dst, _L)] = xy

            def group_body(g, carry):
                base16 = jnp.minimum(g * _L, _BC - _L)
                acc_phase(base16, ps0_v)
                red_phase(base16, ps0_v)
                return carry

            lax.fori_loop(0, ngrp, group_body, 0)

            @pl.when(slot % _WB == _WB - 1)
            def _():
                ebase = (c0 + slot - (_WB - 1)) * _BC
                pltpu.sync_copy(xx_v, xx_hbm.at[pl.ds(ebase, _WB * _BC)])
                pltpu.sync_copy(yy_v, yy_hbm.at[pl.ds(ebase, _WB * _BC)])
                pltpu.sync_copy(xy_v, xy_hbm.at[pl.ds(ebase, _WB * _BC)])

        def step(slot, b):
            bw = (b + 2) % _NB
            ba = (b + 1) % _NB

            @pl.when(slot + 2 < cpw)
            def _():
                fire_w(bw, slot + 2)

            @pl.when(slot + 1 < cpw)
            def _():
                wait(ba, 0)
                fire_a(ba)

            wait(b, 1)
            compute(b, slot)

        fire_w(0, 0)
        wait(0, 0)
        fire_a(0)
        fire_w(1, 1)

        def t_body(tm, carry):
            for b in range(_NB):
                step(tm * _NB + b, b)
            return carry

        lax.fori_loop(0, ntm, t_body, 0)
        for slot in tail_slots:
            step(slot, slot % _NB)

    return sc_dots


def _sqn_body(h_ref, o_ref):
    x = h_ref[...]
    o_ref[...] = jnp.sum(x * x, axis=1, keepdims=True)


@functools.lru_cache(maxsize=None)
def _make_sqn(n_nodes):
    return pl.pallas_call(
        _sqn_body,
        out_shape=jax.ShapeDtypeStruct((n_nodes, 1), jnp.float32),
    )


def _tail_body(xx_ref, yy_ref, xy_ref, o_ref):
    xx = xx_ref[...]
    yy = yy_ref[...]
    xy = xy_ref[...]
    nx = jnp.maximum(jnp.sqrt(xx), _MIN_NORM)
    ny = jnp.maximum(jnp.sqrt(yy), _MIN_NORM)
    fx = jnp.where(nx > _MAXNORM, _MAXNORM / nx, 1.0)
    fy = jnp.where(ny > _MAXNORM, _MAXNORM / ny, 1.0)
    x2 = fx * fx * xx
    y2 = fy * fy * yy
    pxy = fx * fy * xy
    a = 1.0 - 2.0 * _C * pxy + _C * y2
    b = 1.0 - _C * x2
    num2 = a * a * x2 + b * b * y2 - 2.0 * a * b * pxy
    den = jnp.maximum(1.0 - 2.0 * _C * pxy + _C * _C * x2 * y2, _MIN_NORM)
    sqrt_c = _C ** 0.5
    z = sqrt_c * jnp.sqrt(jnp.maximum(num2, 0.0)) / den
    zc = jnp.clip(z, -1.0 + 1e-7, 1.0 - 1e-7)
    dist_c = 0.5 * jnp.log((1.0 + zc) / (1.0 - zc))
    dist = dist_c * 2.0 / sqrt_c
    sqdist = dist * dist
    o_ref[...] = 1.0 / (jnp.exp((sqdist - _R) / _T) + 1.0)


@functools.lru_cache(maxsize=None)
def _make_tail(n_edges):
    rows = n_edges // _D
    return pl.pallas_call(
        _tail_body,
        out_shape=jax.ShapeDtypeStruct((rows, _D), jnp.float32),
    )


def kernel(h, idx):
    n_nodes, d = h.shape
    n_edges = idx.shape[0]
    assert d == _D
    i0 = idx[:, 0]
    i1 = idx[:, 1]
    sqn = _make_sqn(n_nodes)(h).reshape(n_nodes)
    xx, yy, xy = _make_sc_dots(n_nodes, n_edges)(h, i0, i1, sqn)
    probs = _make_tail(n_edges)(
        xx.reshape(-1, _D), yy.reshape(-1, _D), xy.reshape(-1, _D))
    return probs.reshape(n_edges)
